# baseline (speedup 1.0000x reference)
import functools

import jax
import jax.numpy as jnp
from jax import lax
from jax.experimental import pallas as pl
from jax.experimental.pallas import tpu as pltpu
from jax.experimental.pallas import tpu_sc as plsc

EMB_DIM = 32
BATCH = 4096
HIST = 50
TOTAL = BATCH * HIST
NFIELD = 3
VROWS4 = 4000000
NUM_CORES = 2
NUM_SUBCORES = 16
NW = NUM_CORES * NUM_SUBCORES
CHUNK = 128
NBLK_F = TOTAL // CHUNK // NW
K = 10
SUP = K * CHUNK
NSUP_F = NBLK_F // K
JB = BATCH // CHUNK
NTILE = EMB_DIM // 8
OUT_W = HIST * EMB_DIM * BATCH

_mesh = plsc.VectorSubcoreMesh(core_axis_name="c", subcore_axis_name="s")


@functools.partial(
    pl.kernel,
    mesh=_mesh,
    out_type=[jax.ShapeDtypeStruct((OUT_W, 1), jnp.float32)] * NFIELD,
    scratch_types=[
        pltpu.VMEM((NFIELD * NBLK_F, CHUNK), jnp.int32),
        pltpu.VMEM((SUP, EMB_DIM), jnp.float32),
        pltpu.VMEM((SUP, EMB_DIM), jnp.float32),
        pltpu.SemaphoreType.DMA,
        pltpu.SemaphoreType.DMA,
        pltpu.SemaphoreType.DMA,
        pltpu.SemaphoreType.DMA,
    ],
    compiler_params=pltpu.CompilerParams(use_tc_tiling_on_sc=False,
                                         needs_layout_passes=False),
)
def _gather3(qry_hbm, pos_hbm, neg_hbm, table_hbm, out_q, out_p, out_n,
             idx_v, gbuf0, gbuf1, g0, g1, w0, w1):
    wid = lax.axis_index("s") * NUM_CORES + lax.axis_index("c")

    outs = (out_q, out_p, out_n)
    gbuf = (gbuf0, gbuf1)
    gsem = (g0, g1)
    wsem = (w0, w1)

    for f, idx_hbm in enumerate((qry_hbm, pos_hbm, neg_hbm)):
        pltpu.sync_copy(idx_hbm.at[wid], idx_v.at[pl.ds(f * NBLK_F, NBLK_F)])

    def fire(f, u, b):
        for k in range(K):
            pltpu.async_copy(
                table_hbm.at[idx_v.at[f * NBLK_F + u * K + k]],
                gbuf[b].at[pl.ds(k * CHUNK, CHUNK)],
                gsem[b],
            )

    def drain_g(b):
        pltpu.make_async_copy(
            table_hbm.at[pl.ds(0, SUP)], gbuf[b], gsem[b]).wait()

    def drain_w(b):
        pltpu.make_async_copy(
            table_hbm.at[pl.ds(0, SUP)], gbuf[b], wsem[b]).wait()

    def write(f, u, b):
        def body(c, _):
            i = c // 8
            s = c % 8
            for k in range(K):
                m = NBLK_F * wid + u * K + k
                h = m // JB
                jb = m % JB
                dst = (h * NTILE * JB + i * JB + jb) * 1024 + s * CHUNK
                pltpu.async_copy(
                    gbuf[b].at[pl.ds(k * CHUNK, CHUNK), pl.ds(c, 1)],
                    outs[f].at[pl.ds(dst, CHUNK), :],
                    wsem[b],
                )
            return ()

        lax.fori_loop(0, EMB_DIM, body, ())

    def stage(f, u, b, nxt):
        drain_g(b)
        write(f, u, b)
        if nxt is not None:
            drain_w(b)
            fire(f, nxt, b)

    for f in range(NFIELD):
        fire(f, 0, 0)
        fire(f, 1, 1)
        stage(f, 0, 0, 2)
        stage(f, 1, 1, 3)
        stage(f, 2, 0, 4)
        stage(f, 3, 1, None)
        stage(f, 4, 0, None)
        drain_w(1)
        drain_w(0)


def kernel(qry_lkup, pos_lkup, neg_lkup, table):
    table4 = jnp.pad(table, ((0, 0), (0, 128 - EMB_DIM))).reshape(VROWS4,
                                                                  EMB_DIM)
    shaped = lambda a: (a.astype(jnp.int32).T * 4).reshape(NW, NBLK_F, CHUNK)
    outs = _gather3(shaped(qry_lkup), shaped(pos_lkup), shaped(neg_lkup),
                    table4)

    def unpack(flat):
        x = flat.reshape(HIST, NTILE, JB, 8, CHUNK)
        return x.transpose(2, 4, 0, 1, 3).reshape(BATCH, HIST, EMB_DIM)

    return tuple(unpack(o) for o in outs)


# speedup vs baseline: 84.9151x; 84.9151x over previous
import functools

import jax
import jax.numpy as jnp
from jax import lax
from jax.experimental import pallas as pl
from jax.experimental.pallas import tpu as pltpu
from jax.experimental.pallas import tpu_sc as plsc

EMB_DIM = 32
BATCH = 4096
HIST = 50
TOTAL = BATCH * HIST
NFIELD = 3
VROWS4 = 4000000
NUM_CORES = 2
NUM_SUBCORES = 16
NW = NUM_CORES * NUM_SUBCORES
CHUNK = 128
NBLK_F = TOTAL // CHUNK // NW
K = 5
SUP = K * CHUNK
NSUP_F = NBLK_F // K
BLK_W = CHUNK * EMB_DIM
JB = BATCH // CHUNK
NTILE = EMB_DIM // 8
OUT_W = HIST * EMB_DIM * BATCH

_mesh = plsc.VectorSubcoreMesh(core_axis_name="c", subcore_axis_name="s")


@functools.partial(
    pl.kernel,
    mesh=_mesh,
    out_type=[jax.ShapeDtypeStruct((OUT_W,), jnp.float32)] * NFIELD,
    scratch_types=[
        pltpu.VMEM((NFIELD * NBLK_F, CHUNK), jnp.int32),
        pltpu.VMEM((SUP, EMB_DIM), jnp.float32),
        pltpu.VMEM((SUP, EMB_DIM), jnp.float32),
        pltpu.VMEM((K * BLK_W,), jnp.float32),
        pltpu.VMEM((K * BLK_W,), jnp.float32),
        pltpu.SemaphoreType.DMA,
        pltpu.SemaphoreType.DMA,
        pltpu.SemaphoreType.DMA,
        pltpu.SemaphoreType.DMA,
    ],
    compiler_params=pltpu.CompilerParams(use_tc_tiling_on_sc=False,
                                         needs_layout_passes=False),
)
def _gather3(qry_hbm, pos_hbm, neg_hbm, table_hbm, out_q, out_p, out_n,
             idx_v, gbuf0, gbuf1, wbuf0, wbuf1, g0, g1, w0, w1):
    wid = lax.axis_index("s") * NUM_CORES + lax.axis_index("c")

    outs = (out_q, out_p, out_n)
    gbuf = (gbuf0, gbuf1)
    wbuf = (wbuf0, wbuf1)
    gsem = (g0, g1)
    wsem = (w0, w1)

    for f, idx_hbm in enumerate((qry_hbm, pos_hbm, neg_hbm)):
        pltpu.sync_copy(idx_hbm.at[wid], idx_v.at[pl.ds(f * NBLK_F, NBLK_F)])

    lane = lax.broadcasted_iota(jnp.int32, (16,), 0)
    slo = [lane * CHUNK + k * BLK_W for k in range(K)]
    shi = [(lane + 16) * CHUNK + k * BLK_W for k in range(K)]

    def fire(f, u, b):
        for k in range(K):
            pltpu.async_copy(
                table_hbm.at[idx_v.at[f * NBLK_F + u * K + k]],
                gbuf[b].at[pl.ds(k * CHUNK, CHUNK)],
                gsem[b],
            )

    def drain_g(b):
        pltpu.make_async_copy(
            table_hbm.at[pl.ds(0, SUP)], gbuf[b], gsem[b]).wait()

    def drain_w(f, b):
        pltpu.make_async_copy(
            outs[f].at[pl.ds(0, K * BLK_W)], wbuf[b], wsem[b]).wait()

    def transpose(b):
        @functools.partial(plsc.parallel_loop, 0, CHUNK, unroll=4)
        def _(r):
            for k in range(K):
                v0 = gbuf[b][k * CHUNK + r, pl.ds(0, 16)]
                v1 = gbuf[b][k * CHUNK + r, pl.ds(16, 16)]
                plsc.store_scatter(wbuf[b], [slo[k] + r], v0)
                plsc.store_scatter(wbuf[b], [shi[k] + r], v1)

    def write(f, u, b):
        for k in range(K):
            m = NBLK_F * wid + u * K + k
            h = m // JB
            jb = m % JB
            for i in range(NTILE):
                pltpu.async_copy(
                    wbuf[b].at[pl.ds(k * BLK_W + i * 1024, 1024)],
                    outs[f].at[pl.ds((h * NTILE * JB + i * JB + jb) * 1024,
                                     1024)],
                    wsem[b],
                )

    def stage(f, u, b, first):
        drain_g(b)
        if not first:
            drain_w(f, b)
        transpose(b)
        write(f, u, b)

    for f in range(NFIELD):
        fire(f, 0, 0)
        fire(f, 1, 1)
        stage(f, 0, 0, True)
        fire(f, 2, 0)
        stage(f, 1, 1, True)
        fire(f, 3, 1)

        @pl.loop(2, NSUP_F - 2, step=2)
        def _(u):
            stage(f, u, 0, False)
            fire(f, u + 2, 0)
            stage(f, u + 1, 1, False)
            fire(f, u + 3, 1)

        stage(f, NSUP_F - 2, 0, False)
        stage(f, NSUP_F - 1, 1, False)
        drain_w(f, 0)
        drain_w(f, 1)


def kernel(qry_lkup, pos_lkup, neg_lkup, table):
    table4 = jnp.pad(table, ((0, 0), (0, 128 - EMB_DIM))).reshape(VROWS4,
                                                                  EMB_DIM)
    shaped = lambda a: (a.astype(jnp.int32).T * 4).reshape(NW, NBLK_F, CHUNK)
    outs = _gather3(shaped(qry_lkup), shaped(pos_lkup), shaped(neg_lkup),
                    table4)

    def unpack(flat):
        x = flat.reshape(HIST, NTILE, JB, 8, CHUNK)
        return x.transpose(2, 4, 0, 1, 3).reshape(BATCH, HIST, EMB_DIM)

    return tuple(unpack(o) for o in outs)
